# Initial kernel scaffold; baseline (speedup 1.0000x reference)
import jax
import jax.numpy as jnp
from jax.experimental import pallas as pl


def kernel(input1, input2):
    raise NotImplementedError("write your pallas kernel here")


# speedup vs baseline: 1.9179x; 1.9179x over previous
import functools

import jax
import jax.numpy as jnp
from jax import lax
from jax.experimental import pallas as pl
from jax.experimental.pallas import tpu as pltpu
from jax.experimental.pallas import tpu_sc as plsc

H = 128
W = 128
N = H * W
NW = 32
CHUNK = N // NW
L = 16
VECS = CHUNK // L


def _ifloor(x):
    t = x.astype(jnp.int32)
    return t - (t.astype(jnp.float32) > x).astype(jnp.int32)


_mesh = plsc.VectorSubcoreMesh(core_axis_name="c", subcore_axis_name="s")


@functools.partial(
    pl.kernel,
    mesh=_mesh,
    compiler_params=pltpu.CompilerParams(needs_layout_passes=False),
    out_type=jax.ShapeDtypeStruct((N,), jnp.float32),
    scratch_types=[
        pltpu.VMEM((N,), jnp.float32),
        pltpu.VMEM((CHUNK,), jnp.float32),
        pltpu.VMEM((CHUNK,), jnp.float32),
        pltpu.VMEM((CHUNK,), jnp.float32),
    ],
)
def _warp(img_hbm, dh_hbm, dw_hbm, out_hbm, img_v, dh_v, dw_v, out_v):
    wid = lax.axis_index("s") * 2 + lax.axis_index("c")
    base = wid * CHUNK
    pltpu.sync_copy(img_hbm, img_v)
    pltpu.sync_copy(dh_hbm.at[pl.ds(base, CHUNK)], dh_v)
    pltpu.sync_copy(dw_hbm.at[pl.ds(base, CHUNK)], dw_v)

    lane = lax.broadcasted_iota(jnp.int32, (L,), 0)

    def body(it, _):
        off = it * L
        p = base + off
        row = p // W
        col0 = p % W
        hu = dh_v[pl.ds(off, L)] + jnp.broadcast_to(row + 1, (L,)).astype(jnp.float32)
        wu = dw_v[pl.ds(off, L)] + (lane + (col0 + 1)).astype(jnp.float32)
        hf_u = _ifloor(hu)
        wf_u = _ifloor(wu)
        hf = jnp.clip(hf_u, 0, H + 1)
        hc = jnp.clip(hf_u + 1, 0, H + 1)
        wf = jnp.clip(wf_u, 0, W + 1)
        wc = jnp.clip(wf_u + 1, 0, W + 1)
        d_h = hc.astype(jnp.float32) - hu
        d_w = wc.astype(jnp.float32) - wu

        def corner(h, w):
            h0 = h - 1
            w0 = w - 1
            valid = (h0 >= 0) & (h0 < H) & (w0 >= 0) & (w0 < W)
            idx = jnp.clip(h0, 0, H - 1) * W + jnp.clip(w0, 0, W - 1)
            v = plsc.load_gather(img_v, [idx])
            return jnp.where(valid, v, jnp.float32(0.0))

        v00 = corner(hf, wf)
        v10 = corner(hc, wf)
        v01 = corner(hf, wc)
        v11 = corner(hc, wc)
        one = jnp.float32(1.0)
        out = (v00 * (d_w * d_h) + v10 * (d_w * (one - d_h))
               + v01 * ((one - d_w) * d_h) + v11 * ((one - d_w) * (one - d_h)))
        out_v[pl.ds(off, L)] = out
        return 0

    lax.fori_loop(0, VECS, body, 0)
    pltpu.sync_copy(out_v, out_hbm.at[pl.ds(base, CHUNK)])


def kernel(input1, input2):
    img = input1.reshape(N)
    dh = input2[0, 0].reshape(N)
    dw = input2[0, 1].reshape(N)
    out = _warp(img, dh, dw)
    return out.reshape(1, 1, H, W)
